# initial kernel scaffold (unmeasured)
import functools

import jax
import jax.numpy as jnp
from jax import lax
from jax.experimental import pallas as pl
from jax.experimental.pallas import tpu as pltpu

N_DEV = 4
FP8 = jnp.float8_e4m3fn


def kernel(x, w_mat, scale_x, scale_w):
    m_per, k = x.shape
    n_loc = w_mat.shape[1]

    def body(x_ref, w_ref, sx_ref, sw_ref, out_ref,
             comm_ref, w8_ref, send_sems, recv_sems):
        my = lax.axis_index("i")
        left = lax.rem(my + (N_DEV - 1), N_DEV)
        right = lax.rem(my + 1, N_DEV)

        barrier_sem = pltpu.get_barrier_semaphore()
        for nbr in (left, right):
            pl.semaphore_signal(
                barrier_sem, inc=1,
                device_id=(nbr,), device_id_type=pl.DeviceIdType.MESH,
            )
        pl.semaphore_wait(barrier_sem, 2)

        comm_ref[0] = x_ref[...].astype(FP8)
        w8_ref[...] = w_ref[...].astype(FP8)
        s = sx_ref[0] * sw_ref[0]

        out_ref[pl.ds(my * m_per, m_per), :] = (
            jnp.dot(comm_ref[0], w8_ref[...],
                    preferred_element_type=jnp.float32) * s
        )

        for h in range(N_DEV - 1):
            rdma = pltpu.make_async_remote_copy(
                src_ref=comm_ref.at[h],
                dst_ref=comm_ref.at[h + 1],
                send_sem=send_sems.at[h],
                recv_sem=recv_sems.at[h + 1],
                device_id=(right,),
                device_id_type=pl.DeviceIdType.MESH,
            )
            rdma.start()
            rdma.wait()

            origin = lax.rem(my + (N_DEV - 1 - h), N_DEV)
            out_ref[pl.ds(origin * m_per, m_per), :] = (
                jnp.dot(comm_ref[h + 1], w8_ref[...],
                        preferred_element_type=jnp.float32) * s
            )

    return pl.pallas_call(
        body,
        out_shape=jax.ShapeDtypeStruct((N_DEV * m_per, n_loc), jnp.float32),
        in_specs=[
            pl.BlockSpec(memory_space=pltpu.VMEM),
            pl.BlockSpec(memory_space=pltpu.VMEM),
            pl.BlockSpec(memory_space=pltpu.SMEM),
            pl.BlockSpec(memory_space=pltpu.SMEM),
        ],
        out_specs=pl.BlockSpec(memory_space=pltpu.VMEM),
        scratch_shapes=[
            pltpu.VMEM((N_DEV, m_per, k), FP8),
            pltpu.VMEM((k, n_loc), FP8),
            pltpu.SemaphoreType.DMA((N_DEV,)),
            pltpu.SemaphoreType.DMA((N_DEV,)),
        ],
        compiler_params=pltpu.CompilerParams(collective_id=0),
    )(x, w_mat, scale_x, scale_w)


# baseline (device time: 167881 ns/iter reference)
import jax
import jax.numpy as jnp
from jax import lax
from jax.experimental import pallas as pl
from jax.experimental.pallas import tpu as pltpu

N_DEV = 4
FP8 = jnp.float8_e4m3fn
N_SLOT = 3


def kernel(x, w_mat, scale_x, scale_w):
    m_per, k = x.shape
    n_loc = w_mat.shape[1]

    x8 = x.astype(FP8)
    w8 = w_mat.astype(FP8)

    def body(x_ref, w_ref, sx_ref, sw_ref, out_ref,
             comm_ref, send_sems, recv_sems):
        my = lax.axis_index("i")
        left = lax.rem(my + (N_DEV - 1), N_DEV)
        right = lax.rem(my + 1, N_DEV)

        barrier_sem = pltpu.get_barrier_semaphore()
        for nbr in (left, right):
            pl.semaphore_signal(
                barrier_sem, inc=1,
                device_id=(nbr,), device_id_type=pl.DeviceIdType.MESH,
            )
        pl.semaphore_wait(barrier_sem, 2)

        comm_ref[0] = x_ref[...]
        s = sx_ref[0] * sw_ref[0]

        out_ref[pl.ds(my * m_per, m_per), :] = (
            jnp.dot(x_ref[...], w_ref[...],
                    preferred_element_type=jnp.float32) * s
        )

        for h in range(N_DEV - 1):
            rdma = pltpu.make_async_remote_copy(
                src_ref=comm_ref.at[h % N_SLOT],
                dst_ref=comm_ref.at[(h + 1) % N_SLOT],
                send_sem=send_sems.at[h],
                recv_sem=recv_sems.at[h],
                device_id=(right,),
                device_id_type=pl.DeviceIdType.MESH,
            )
            rdma.start()
            rdma.wait()

            origin = lax.rem(my + (N_DEV - 1 - h), N_DEV)
            out_ref[pl.ds(origin * m_per, m_per), :] = (
                jnp.dot(comm_ref[(h + 1) % N_SLOT], w_ref[...],
                        preferred_element_type=jnp.float32) * s
            )

    out = pl.pallas_call(
        body,
        out_shape=jax.ShapeDtypeStruct((N_DEV * m_per, n_loc), jnp.float32),
        in_specs=[
            pl.BlockSpec(memory_space=pltpu.VMEM),
            pl.BlockSpec(memory_space=pltpu.VMEM),
            pl.BlockSpec(memory_space=pltpu.SMEM),
            pl.BlockSpec(memory_space=pltpu.SMEM),
        ],
        out_specs=pl.BlockSpec(memory_space=pltpu.VMEM),
        scratch_shapes=[
            pltpu.VMEM((N_SLOT, m_per, k), FP8),
            pltpu.SemaphoreType.DMA((N_DEV - 1,)),
            pltpu.SemaphoreType.DMA((N_DEV - 1,)),
        ],
        compiler_params=pltpu.CompilerParams(collective_id=0),
    )(x8, w8, scale_x, scale_w)
    return out


# device time: 93081 ns/iter; 1.8036x vs baseline; 1.8036x over previous
import jax
import jax.numpy as jnp
from jax import lax
from jax.experimental import pallas as pl
from jax.experimental.pallas import tpu as pltpu

N_DEV = 4
FP8 = jnp.float8_e4m3fn
N_SLOT = 3


def kernel(x, w_mat, scale_x, scale_w):
    m_per, k = x.shape
    n_loc = w_mat.shape[1]
    m_half = m_per // 2

    x8 = x.astype(FP8)
    w8 = w_mat.astype(FP8)

    def body(x_ref, w_ref, sx_ref, sw_ref, out_ref,
             cw_ref, ccw_ref, cw_ssem, cw_rsem, ccw_ssem, ccw_rsem):
        my = lax.axis_index("i")
        left = lax.rem(my + (N_DEV - 1), N_DEV)
        right = lax.rem(my + 1, N_DEV)

        barrier_sem = pltpu.get_barrier_semaphore()
        for nbr in (left, right):
            pl.semaphore_signal(
                barrier_sem, inc=1,
                device_id=(nbr,), device_id_type=pl.DeviceIdType.MESH,
            )
        pl.semaphore_wait(barrier_sem, 2)

        cw_ref[0] = x_ref[pl.ds(0, m_half), :]
        ccw_ref[0] = x_ref[pl.ds(m_half, m_half), :]
        s = sx_ref[0] * sw_ref[0]

        def make_hop(h):
            cw = pltpu.make_async_remote_copy(
                src_ref=cw_ref.at[h % N_SLOT],
                dst_ref=cw_ref.at[(h + 1) % N_SLOT],
                send_sem=cw_ssem.at[h],
                recv_sem=cw_rsem.at[h],
                device_id=(right,),
                device_id_type=pl.DeviceIdType.MESH,
            )
            ccw = pltpu.make_async_remote_copy(
                src_ref=ccw_ref.at[h % N_SLOT],
                dst_ref=ccw_ref.at[(h + 1) % N_SLOT],
                send_sem=ccw_ssem.at[h],
                recv_sem=ccw_rsem.at[h],
                device_id=(left,),
                device_id_type=pl.DeviceIdType.MESH,
            )
            return cw, ccw

        cw, ccw = make_hop(0)
        cw.start()
        ccw.start()

        out_ref[pl.ds(my * m_per, m_per), :] = (
            jnp.dot(x_ref[...], w_ref[...],
                    preferred_element_type=jnp.float32) * s
        )

        for h in range(N_DEV - 1):
            cw.wait_recv()
            ccw.wait_recv()
            if h < N_DEV - 2:
                cw.wait_send()
                ccw.wait_send()
                cw, ccw = make_hop(h + 1)
                cw.start()
                ccw.start()
            o_cw = lax.rem(my + (N_DEV - 1 - h), N_DEV)
            out_ref[pl.ds(o_cw * m_per, m_half), :] = (
                jnp.dot(cw_ref[(h + 1) % N_SLOT], w_ref[...],
                        preferred_element_type=jnp.float32) * s
            )
            o_ccw = lax.rem(my + h + 1, N_DEV)
            out_ref[pl.ds(o_ccw * m_per + m_half, m_half), :] = (
                jnp.dot(ccw_ref[(h + 1) % N_SLOT], w_ref[...],
                        preferred_element_type=jnp.float32) * s
            )
        cw.wait_send()
        ccw.wait_send()

    return pl.pallas_call(
        body,
        out_shape=jax.ShapeDtypeStruct((N_DEV * m_per, n_loc), jnp.float32),
        in_specs=[
            pl.BlockSpec(memory_space=pltpu.VMEM),
            pl.BlockSpec(memory_space=pltpu.VMEM),
            pl.BlockSpec(memory_space=pltpu.SMEM),
            pl.BlockSpec(memory_space=pltpu.SMEM),
        ],
        out_specs=pl.BlockSpec(memory_space=pltpu.VMEM),
        scratch_shapes=[
            pltpu.VMEM((N_SLOT, m_half, k), FP8),
            pltpu.VMEM((N_SLOT, m_half, k), FP8),
            pltpu.SemaphoreType.DMA((N_DEV - 1,)),
            pltpu.SemaphoreType.DMA((N_DEV - 1,)),
            pltpu.SemaphoreType.DMA((N_DEV - 1,)),
            pltpu.SemaphoreType.DMA((N_DEV - 1,)),
        ],
        compiler_params=pltpu.CompilerParams(collective_id=0),
    )(x8, w8, scale_x, scale_w)


# device time: 92736 ns/iter; 1.8103x vs baseline; 1.0037x over previous
import jax
import jax.numpy as jnp
from jax import lax
from jax.experimental import pallas as pl
from jax.experimental.pallas import tpu as pltpu

N_DEV = 4
FP8 = jnp.float8_e4m3fn
N_SLOT = 3


def kernel(x, w_mat, scale_x, scale_w):
    m_per, k = x.shape
    n_loc = w_mat.shape[1]
    m_half = m_per // 2

    x8 = x.astype(FP8)
    w8 = w_mat.astype(FP8)

    def body(x_ref, w_ref, sx_ref, sw_ref, out_ref,
             cw_ref, ccw_ref, cw_ssem, cw_rsem, ccw_ssem, ccw_rsem):
        my = lax.axis_index("i")
        left = lax.rem(my + (N_DEV - 1), N_DEV)
        right = lax.rem(my + 1, N_DEV)

        barrier_sem = pltpu.get_barrier_semaphore()
        for nbr in (left, right):
            pl.semaphore_signal(
                barrier_sem, inc=1,
                device_id=(nbr,), device_id_type=pl.DeviceIdType.MESH,
            )
        pl.semaphore_wait(barrier_sem, 2)

        s = sx_ref[0] * sw_ref[0]

        def make_hop(h):
            cw_src = (x_ref.at[pl.ds(0, m_half), :] if h == 0
                      else cw_ref.at[h % N_SLOT])
            ccw_src = (x_ref.at[pl.ds(m_half, m_half), :] if h == 0
                       else ccw_ref.at[h % N_SLOT])
            cw = pltpu.make_async_remote_copy(
                src_ref=cw_src,
                dst_ref=cw_ref.at[(h + 1) % N_SLOT],
                send_sem=cw_ssem.at[h],
                recv_sem=cw_rsem.at[h],
                device_id=(right,),
                device_id_type=pl.DeviceIdType.MESH,
            )
            ccw = pltpu.make_async_remote_copy(
                src_ref=ccw_src,
                dst_ref=ccw_ref.at[(h + 1) % N_SLOT],
                send_sem=ccw_ssem.at[h],
                recv_sem=ccw_rsem.at[h],
                device_id=(left,),
                device_id_type=pl.DeviceIdType.MESH,
            )
            return cw, ccw

        cw, ccw = make_hop(0)
        cw.start()
        ccw.start()

        out_ref[pl.ds(my * m_per, m_per), :] = (
            jnp.dot(x_ref[...], w_ref[...],
                    preferred_element_type=jnp.float32) * s
        )

        for h in range(N_DEV - 1):
            cw.wait_recv()
            ccw.wait_recv()
            if h < N_DEV - 2:
                cw.wait_send()
                ccw.wait_send()
                cw, ccw = make_hop(h + 1)
                cw.start()
                ccw.start()
            o_cw = lax.rem(my + (N_DEV - 1 - h), N_DEV)
            out_ref[pl.ds(o_cw * m_per, m_half), :] = (
                jnp.dot(cw_ref[(h + 1) % N_SLOT], w_ref[...],
                        preferred_element_type=jnp.float32) * s
            )
            o_ccw = lax.rem(my + h + 1, N_DEV)
            out_ref[pl.ds(o_ccw * m_per + m_half, m_half), :] = (
                jnp.dot(ccw_ref[(h + 1) % N_SLOT], w_ref[...],
                        preferred_element_type=jnp.float32) * s
            )
        cw.wait_send()
        ccw.wait_send()

    return pl.pallas_call(
        body,
        out_shape=jax.ShapeDtypeStruct((N_DEV * m_per, n_loc), jnp.float32),
        in_specs=[
            pl.BlockSpec(memory_space=pltpu.VMEM),
            pl.BlockSpec(memory_space=pltpu.VMEM),
            pl.BlockSpec(memory_space=pltpu.SMEM),
            pl.BlockSpec(memory_space=pltpu.SMEM),
        ],
        out_specs=pl.BlockSpec(memory_space=pltpu.VMEM),
        scratch_shapes=[
            pltpu.VMEM((N_SLOT, m_half, k), FP8),
            pltpu.VMEM((N_SLOT, m_half, k), FP8),
            pltpu.SemaphoreType.DMA((N_DEV - 1,)),
            pltpu.SemaphoreType.DMA((N_DEV - 1,)),
            pltpu.SemaphoreType.DMA((N_DEV - 1,)),
            pltpu.SemaphoreType.DMA((N_DEV - 1,)),
        ],
        compiler_params=pltpu.CompilerParams(collective_id=0),
    )(x8, w8, scale_x, scale_w)


# device time: 84704 ns/iter; 1.9820x vs baseline; 1.0948x over previous
import jax
import jax.numpy as jnp
from jax import lax
from jax.experimental import pallas as pl
from jax.experimental.pallas import tpu as pltpu

N_DEV = 4
FP8 = jnp.float8_e4m3fn
N_SLOT = 3
SUB = 256


def kernel(x, w_mat, scale_x, scale_w):
    m_per, k = x.shape
    n_loc = w_mat.shape[1]
    m_half = m_per // 2

    def body(x_ref, w_ref, sx_ref, sw_ref, out_ref,
             cw_ref, ccw_ref, w8_ref, xstage, wstage, ostage,
             cw_ssem, cw_rsem, ccw_ssem, ccw_rsem, xsem, wsem, osem):
        my = lax.axis_index("i")
        left = lax.rem(my + (N_DEV - 1), N_DEV)
        right = lax.rem(my + 1, N_DEV)

        barrier_sem = pltpu.get_barrier_semaphore()
        for nbr in (left, right):
            pl.semaphore_signal(
                barrier_sem, inc=1,
                device_id=(nbr,), device_id_type=pl.DeviceIdType.MESH,
            )
        pl.semaphore_wait(barrier_sem, 2)

        def hop(direction, h, s):
            ref = cw_ref if direction == 0 else ccw_ref
            ssem = cw_ssem if direction == 0 else ccw_ssem
            rsem = cw_rsem if direction == 0 else ccw_rsem
            return pltpu.make_async_remote_copy(
                src_ref=ref.at[h % N_SLOT, pl.ds(s * SUB, SUB), :],
                dst_ref=ref.at[(h + 1) % N_SLOT, pl.ds(s * SUB, SUB), :],
                send_sem=ssem.at[h, s],
                recv_sem=rsem.at[h, s],
                device_id=(right if direction == 0 else left,),
                device_id_type=pl.DeviceIdType.MESH,
            )

        pieces = [
            (cw_ref, 0, 0, 0, 0),
            (ccw_ref, m_half, 0, 1, 1),
            (cw_ref, SUB, SUB, 0, 0),
            (ccw_ref, m_half + SUB, SUB, 1, 1),
        ]
        copies = []
        for i, (_, xr, _, buf, _) in enumerate(pieces[:2]):
            cp = pltpu.make_async_copy(
                x_ref.at[pl.ds(xr, SUB), :], xstage.at[buf], xsem.at[buf])
            cp.start()
            copies.append(cp)
        for i, (dref, xr, dr, buf, d) in enumerate(pieces):
            copies[i].wait()
            dref[0, pl.ds(dr, SUB), :] = xstage[buf].astype(FP8)
            hop(d, 0, dr // SUB).start()
            if i + 2 < len(pieces):
                nxt = pieces[i + 2]
                cp = pltpu.make_async_copy(
                    x_ref.at[pl.ds(nxt[1], SUB), :],
                    xstage.at[nxt[3]], xsem.at[nxt[3]])
                cp.start()
                copies.append(cp)

        w_piece = k // 4
        wcopies = []
        for p in range(2):
            cp = pltpu.make_async_copy(
                w_ref.at[pl.ds(p * w_piece, w_piece), :],
                wstage.at[p], wsem.at[p])
            cp.start()
            wcopies.append(cp)
        for p in range(4):
            wcopies[p].wait()
            w8_ref[pl.ds(p * w_piece, w_piece), :] = wstage[p % 2].astype(FP8)
            if p + 2 < 4:
                cp = pltpu.make_async_copy(
                    w_ref.at[pl.ds((p + 2) * w_piece, w_piece), :],
                    wstage.at[p % 2], wsem.at[p % 2])
                cp.start()
                wcopies.append(cp)

        s_scale = sx_ref[0] * sw_ref[0]
        pending = [None, None]
        emit_n = [0]

        def emit(block_fp8, out_row):
            buf = emit_n[0] % 2
            emit_n[0] += 1
            if pending[buf] is not None:
                pending[buf].wait()
            ostage[buf] = jnp.dot(
                block_fp8, w8_ref[...],
                preferred_element_type=jnp.float32) * s_scale
            cp = pltpu.make_async_copy(
                ostage.at[buf], out_ref.at[pl.ds(out_row, SUB), :],
                osem.at[buf])
            cp.start()
            pending[buf] = cp

        for sub in range(2):
            emit(cw_ref[0, pl.ds(sub * SUB, SUB), :],
                 my * m_per + sub * SUB)
        for sub in range(2):
            emit(ccw_ref[0, pl.ds(sub * SUB, SUB), :],
                 my * m_per + m_half + sub * SUB)

        last_sends = []
        for h in range(N_DEV - 1):
            for sub in range(2):
                for d in range(2):
                    cur = hop(d, h, sub)
                    cur.wait_recv()
                    if h < N_DEV - 2:
                        cur.wait_send()
                        hop(d, h + 1, sub).start()
                    else:
                        last_sends.append(cur)
                o_cw = lax.rem(my + (N_DEV - 1 - h), N_DEV)
                emit(cw_ref[(h + 1) % N_SLOT, pl.ds(sub * SUB, SUB), :],
                     o_cw * m_per + sub * SUB)
                o_ccw = lax.rem(my + h + 1, N_DEV)
                emit(ccw_ref[(h + 1) % N_SLOT, pl.ds(sub * SUB, SUB), :],
                     o_ccw * m_per + m_half + sub * SUB)

        for snd in last_sends:
            snd.wait_send()
        for cp in pending:
            if cp is not None:
                cp.wait()

    return pl.pallas_call(
        body,
        out_shape=jax.ShapeDtypeStruct((N_DEV * m_per, n_loc), jnp.float32),
        in_specs=[
            pl.BlockSpec(memory_space=pl.ANY),
            pl.BlockSpec(memory_space=pl.ANY),
            pl.BlockSpec(memory_space=pltpu.SMEM),
            pl.BlockSpec(memory_space=pltpu.SMEM),
        ],
        out_specs=pl.BlockSpec(memory_space=pl.ANY),
        scratch_shapes=[
            pltpu.VMEM((N_SLOT, m_half, k), FP8),
            pltpu.VMEM((N_SLOT, m_half, k), FP8),
            pltpu.VMEM((k, n_loc), FP8),
            pltpu.VMEM((2, SUB, k), jnp.float32),
            pltpu.VMEM((2, k // 4, n_loc), jnp.float32),
            pltpu.VMEM((2, SUB, n_loc), jnp.float32),
            pltpu.SemaphoreType.DMA((N_DEV - 1, 2)),
            pltpu.SemaphoreType.DMA((N_DEV - 1, 2)),
            pltpu.SemaphoreType.DMA((N_DEV - 1, 2)),
            pltpu.SemaphoreType.DMA((N_DEV - 1, 2)),
            pltpu.SemaphoreType.DMA((2,)),
            pltpu.SemaphoreType.DMA((2,)),
            pltpu.SemaphoreType.DMA((2,)),
        ],
        compiler_params=pltpu.CompilerParams(collective_id=0),
    )(x, w_mat, scale_x, scale_w)


# device time: 81184 ns/iter; 2.0679x vs baseline; 1.0434x over previous
import jax
import jax.numpy as jnp
from jax import lax
from jax.experimental import pallas as pl
from jax.experimental.pallas import tpu as pltpu

N_DEV = 4
FP8 = jnp.float8_e4m3fn
QTR = 128
NQ = 8
NF = 4


def kernel(x, w_mat, scale_x, scale_w):
    m_per, k = x.shape
    n_loc = w_mat.shape[1]
    m_half = m_per // 2

    def body(x_ref, w_ref, sx_ref, sw_ref, out_ref,
             x8_ref, nbrL, nbrR, diag, w8_ref, xstage, wstage, ostage,
             toL_s, toL_r, toR_s, toR_r, fwdR_s, fwdR_r, fwdL_s, fwdL_r,
             xsem, wsem, osem):
        my = lax.axis_index("i")
        left = lax.rem(my + (N_DEV - 1), N_DEV)
        right = lax.rem(my + 1, N_DEV)

        def toL(q):
            return pltpu.make_async_remote_copy(
                src_ref=x8_ref.at[pl.ds(q * QTR, QTR), :],
                dst_ref=nbrR.at[pl.ds(q * QTR, QTR), :],
                send_sem=toL_s.at[q], recv_sem=toL_r.at[q],
                device_id=(left,), device_id_type=pl.DeviceIdType.MESH,
            )

        def toR(q):
            return pltpu.make_async_remote_copy(
                src_ref=x8_ref.at[pl.ds(q * QTR, QTR), :],
                dst_ref=nbrL.at[pl.ds(q * QTR, QTR), :],
                send_sem=toR_s.at[q], recv_sem=toR_r.at[q],
                device_id=(right,), device_id_type=pl.DeviceIdType.MESH,
            )

        def fwdR(q):
            return pltpu.make_async_remote_copy(
                src_ref=nbrL.at[pl.ds(q * QTR, QTR), :],
                dst_ref=diag.at[pl.ds(q * QTR, QTR), :],
                send_sem=fwdR_s.at[q], recv_sem=fwdR_r.at[q],
                device_id=(right,), device_id_type=pl.DeviceIdType.MESH,
            )

        def fwdL(q):
            return pltpu.make_async_remote_copy(
                src_ref=nbrR.at[pl.ds(m_half + q * QTR, QTR), :],
                dst_ref=diag.at[pl.ds(m_half + q * QTR, QTR), :],
                send_sem=fwdL_s.at[q], recv_sem=fwdL_r.at[q],
                device_id=(left,), device_id_type=pl.DeviceIdType.MESH,
            )

        def xdma(q, buf):
            return pltpu.make_async_copy(
                x_ref.at[pl.ds(q * QTR, QTR), :], xstage.at[buf],
                xsem.at[buf])

        xdma(0, 0).start()
        xdma(1, 1).start()

        barrier_sem = pltpu.get_barrier_semaphore()
        for nbr in (left, right):
            pl.semaphore_signal(
                barrier_sem, inc=1,
                device_id=(nbr,), device_id_type=pl.DeviceIdType.MESH,
            )
        pl.semaphore_wait(barrier_sem, 2)

        for q in range(NQ):
            buf = q % 2
            xdma(q, buf).wait()
            x8_ref[pl.ds(q * QTR, QTR), :] = xstage[buf].astype(FP8)
            toL(q).start()
            toR(q).start()
            if q + 2 < NQ:
                xdma(q + 2, buf).start()

        w_piece = k // 4
        for p in range(2):
            pltpu.make_async_copy(
                w_ref.at[pl.ds(p * w_piece, w_piece), :],
                wstage.at[p], wsem.at[p]).start()
        for p in range(4):
            buf = p % 2
            pltpu.make_async_copy(
                w_ref.at[pl.ds(p * w_piece, w_piece), :],
                wstage.at[buf], wsem.at[buf]).wait()
            w8_ref[pl.ds(p * w_piece, w_piece), :] = wstage[buf].astype(FP8)
            if p + 2 < 4:
                pltpu.make_async_copy(
                    w_ref.at[pl.ds((p + 2) * w_piece, w_piece), :],
                    wstage.at[buf], wsem.at[buf]).start()

        s_scale = sx_ref[0] * sw_ref[0]
        pending = [None, None]
        emit_n = [0]

        def emit(block_fp8, out_row):
            buf = emit_n[0] % 2
            emit_n[0] += 1
            if pending[buf] is not None:
                pending[buf].wait()
            ostage[buf] = jnp.dot(
                block_fp8, w8_ref[...],
                preferred_element_type=jnp.float32) * s_scale
            cp = pltpu.make_async_copy(
                ostage.at[buf], out_ref.at[pl.ds(out_row, QTR), :],
                osem.at[buf])
            cp.start()
            pending[buf] = cp

        for q in range(NQ):
            emit(x8_ref[pl.ds(q * QTR, QTR), :], my * m_per + q * QTR)

        for q in range(NQ):
            rL = toR(q)
            rL.wait_recv()
            if q < NF:
                rL.wait_send()
                fwdR(q).start()
            emit(nbrL[pl.ds(q * QTR, QTR), :], left * m_per + q * QTR)

            rR = toL(q)
            rR.wait_recv()
            if q >= NQ - NF:
                rR.wait_send()
                fwdL(q - NF).start()
            emit(nbrR[pl.ds(q * QTR, QTR), :], right * m_per + q * QTR)

        o_diag = lax.rem(my + 2, N_DEV)
        for q in range(NF):
            fR = fwdR(q)
            fR.wait_recv()
            emit(diag[pl.ds(q * QTR, QTR), :], o_diag * m_per + q * QTR)
            fL = fwdL(q)
            fL.wait_recv()
            emit(diag[pl.ds(m_half + q * QTR, QTR), :],
                 o_diag * m_per + m_half + q * QTR)

        for q in range(NQ):
            if q >= NF:
                toR(q).wait_send()
            if q < NQ - NF:
                toL(q).wait_send()
        for q in range(NF):
            fwdR(q).wait_send()
            fwdL(q).wait_send()
        for cp in pending:
            if cp is not None:
                cp.wait()

    return pl.pallas_call(
        body,
        out_shape=jax.ShapeDtypeStruct((N_DEV * m_per, n_loc), jnp.float32),
        in_specs=[
            pl.BlockSpec(memory_space=pl.ANY),
            pl.BlockSpec(memory_space=pl.ANY),
            pl.BlockSpec(memory_space=pltpu.SMEM),
            pl.BlockSpec(memory_space=pltpu.SMEM),
        ],
        out_specs=pl.BlockSpec(memory_space=pl.ANY),
        scratch_shapes=[
            pltpu.VMEM((m_per, k), FP8),
            pltpu.VMEM((m_per, k), FP8),
            pltpu.VMEM((m_per, k), FP8),
            pltpu.VMEM((m_per, k), FP8),
            pltpu.VMEM((k, n_loc), FP8),
            pltpu.VMEM((2, QTR, k), jnp.float32),
            pltpu.VMEM((2, k // 4, n_loc), jnp.float32),
            pltpu.VMEM((2, QTR, n_loc), jnp.float32),
            pltpu.SemaphoreType.DMA((NQ,)),
            pltpu.SemaphoreType.DMA((NQ,)),
            pltpu.SemaphoreType.DMA((NQ,)),
            pltpu.SemaphoreType.DMA((NQ,)),
            pltpu.SemaphoreType.DMA((NF,)),
            pltpu.SemaphoreType.DMA((NF,)),
            pltpu.SemaphoreType.DMA((NF,)),
            pltpu.SemaphoreType.DMA((NF,)),
            pltpu.SemaphoreType.DMA((2,)),
            pltpu.SemaphoreType.DMA((2,)),
            pltpu.SemaphoreType.DMA((2,)),
        ],
        compiler_params=pltpu.CompilerParams(collective_id=0),
    )(x, w_mat, scale_x, scale_w)
